# baseline (speedup 1.0000x reference)
import functools

import jax
import jax.numpy as jnp
from jax import lax
from jax.experimental import pallas as pl
from jax.experimental.pallas import tpu as pltpu
from jax.experimental.pallas import tpu_sc as plsc

L = 16
D = 32
B = 16384
NC = 2
NS = 16
NW = NC * NS
BPW = B // NW
NBLK = 7813
BPT = 245
CBLK = 8
CHK = CBLK * 128
NCH = 31
MAXB = NBLK - CBLK
LCAP = 2048
SENT = 0x7FFFFFF


def _phase_a(uid_hbm, iid_hbm, ut_hbm, it_hbm,
             u0_hbm, u1_hbm, i0_hbm, i1_hbm,
             ids_v, chunk_v, lid_v, lpos_v, col_v, zero_v,
             sh_v, sem):
    cid = lax.axis_index("c")
    sid = lax.axis_index("s")
    wid = sid * NC + cid
    iota = lax.iota(jnp.int32, L)

    def zf(j, c):
        zero_v[pl.ds(j * L, L)] = jnp.zeros((L,), jnp.float32)
        return c

    lax.fori_loop(0, 512, zf, 0)
    myo = sid * (B * D // NS)

    lo = wid * (BPT * 128)
    hi = lo + BPT * 128
    d_lo = iota
    d_hi = iota + L

    sz = B * D // NS
    for tab, idsrc, o0, o1 in ((ut_hbm, uid_hbm, u0_hbm, u1_hbm),
                               (it_hbm, iid_hbm, i0_hbm, i1_hbm)):
        sh = sh_v
        for q in range(4):
            pltpu.sync_copy(zero_v, sh.at[pl.ds(myo + q * 8192, 8192)])
        plsc.subcore_barrier()

        pltpu.sync_copy(idsrc, ids_v)

        def pre(j, c):
            lid_v[pl.ds(j * L, L)] = jnp.zeros((L,), jnp.int32) + SENT
            return c

        lax.fori_loop(0, LCAP // L, pre, 0)

        def build(v, cur):
            vec = ids_v[pl.ds(v * L, L)]
            m = (vec >= lo) & (vec < hi)
            mi = jnp.where(m, 1, 0).astype(jnp.int32)
            cs = plsc.cumsum(mi)
            slots = cur + cs - mi
            plsc.store_scatter(lid_v, [slots], vec, mask=m)
            plsc.store_scatter(lpos_v, [slots], v * L + iota, mask=m)
            return cur + lax.reduce_max(cs, (0,))

        n = lax.fori_loop(0, B // L, build, 0)
        nv = (n + L - 1) // L

        def blk_of(c):
            return lax.min(wid * BPT + c * CBLK, MAXB)

        def fire(c):
            off = pl.multiple_of(blk_of(c) * 128, 128)
            pltpu.async_copy(tab.at[:, pl.ds(off, CHK)],
                             chunk_v.at[c % 2], sem)

        fire(0)

        def chunk_loop(c, carry):
            @pl.when(c + 1 < NCH)
            def _():
                fire(c + 1)

            pltpu.make_async_copy(tab.at[:, pl.ds(0, CHK)],
                                  chunk_v.at[0], sem).wait()
            clo = blk_of(c) * 128
            chi = clo + CHK
            parb = jnp.zeros((L,), jnp.int32) + (c % 2)

            def scan(j, c2):
                lvec = lid_v[pl.ds(j * L, L)]
                pvec = lpos_v[pl.ds(j * L, L)]
                m0 = (lvec >= clo) & (lvec < chi)

                def cond(m):
                    return lax.reduce_max(
                        plsc.all_reduce_population_count(m), (0,)) > 0

                def body(m):
                    l = lax.reduce_max(plsc.all_reduce_ffs(m), (0,))
                    sel = iota == l
                    idq = lax.reduce_max(jnp.where(sel, lvec, 0), (0,)) - clo
                    pos = lax.reduce_max(jnp.where(sel, pvec, 0), (0,))
                    qb = jnp.zeros((L,), jnp.int32) + idq
                    v_lo = plsc.load_gather(chunk_v, [parb, d_lo, qb])
                    v_hi = plsc.load_gather(chunk_v, [parb, d_hi, qb])
                    col_v[pl.ds(0, L)] = v_lo
                    col_v[pl.ds(L, L)] = v_hi
                    pltpu.sync_copy(col_v, sh.at[pl.ds(pos * D, D)])
                    return m & jnp.logical_not(sel)

                lax.while_loop(cond, body, m0)
                return c2

            lax.fori_loop(0, nv, scan, 0)
            return carry

        lax.fori_loop(0, NCH, chunk_loop, 0)

        plsc.subcore_barrier()

        @pl.when(cid == 0)
        def _():
            pltpu.sync_copy(sh.at[pl.ds(myo, sz)], o0.at[pl.ds(myo, sz)])

        @pl.when(cid == 1)
        def _():
            pltpu.sync_copy(sh.at[pl.ds(myo, sz)], o1.at[pl.ds(myo, sz)])


def _phase_b(u0_hbm, u1_hbm, i0_hbm, i1_hbm, out_hbm,
             a_v, b_v, c_v, out_v):
    wid = lax.axis_index("s") * NC + lax.axis_index("c")
    base = wid * BPW
    base32 = base * D
    n = BPW * D
    iota = lax.iota(jnp.int32, L)

    pltpu.sync_copy(u0_hbm.at[pl.ds(base32, n)], a_v)
    pltpu.sync_copy(u1_hbm.at[pl.ds(base32, n)], b_v)

    def addu(j, c):
        a_v[pl.ds(j * L, L)] = a_v[pl.ds(j * L, L)] + b_v[pl.ds(j * L, L)]
        return c

    lax.fori_loop(0, n // L, addu, 0)

    pltpu.sync_copy(i0_hbm.at[pl.ds(base32, n)], b_v)
    pltpu.sync_copy(i1_hbm.at[pl.ds(base32, n)], c_v)

    def prod(j, c):
        a_v[pl.ds(j * L, L)] = a_v[pl.ds(j * L, L)] * (
            b_v[pl.ds(j * L, L)] + c_v[pl.ds(j * L, L)])
        return c

    lax.fori_loop(0, n // L, prod, 0)

    def group(g, carry):
        acc = jnp.zeros((L,), jnp.float32)
        rowbase = (g * L + iota) * D
        for d in range(D):
            col = lax.rem(iota + d, D)
            acc = acc + plsc.load_gather(a_v, [rowbase + col])
        out_v[pl.ds(g * L, L)] = acc
        return carry

    lax.fori_loop(0, BPW // L, group, 0)

    pltpu.sync_copy(out_v, out_hbm.at[pl.ds(base, BPW)])


@jax.jit
def _run(user_ids, item_ids, user_table_t, item_table_t):
    mesh = plsc.VectorSubcoreMesh(core_axis_name="c", subcore_axis_name="s")
    part = jax.ShapeDtypeStruct((B * D,), jnp.float32)
    ka = pl.kernel(
        _phase_a,
        out_type=[part, part, part, part],
        mesh=mesh,
        compiler_params=pltpu.CompilerParams(needs_layout_passes=False),
        scratch_types=[
            pltpu.VMEM((B,), jnp.int32),
            pltpu.VMEM((2, D, CHK), jnp.float32),
            pltpu.VMEM((LCAP,), jnp.int32),
            pltpu.VMEM((LCAP,), jnp.int32),
            pltpu.VMEM((D,), jnp.float32),
            pltpu.VMEM((8192,), jnp.float32),
            pltpu.VMEM_SHARED((B * D,), jnp.float32),
            pltpu.SemaphoreType.DMA,
        ],
    )
    u0, u1, i0, i1 = ka(user_ids, item_ids, user_table_t, item_table_t)
    kb = pl.kernel(
        _phase_b,
        out_type=jax.ShapeDtypeStruct((B,), jnp.float32),
        mesh=mesh,
        compiler_params=pltpu.CompilerParams(needs_layout_passes=False),
        scratch_types=[
            pltpu.VMEM((BPW * D,), jnp.float32),
            pltpu.VMEM((BPW * D,), jnp.float32),
            pltpu.VMEM((BPW * D,), jnp.float32),
            pltpu.VMEM((BPW,), jnp.float32),
        ],
    )
    return kb(u0, u1, i0, i1)


def kernel(user_ids, item_ids, user_table, item_table):
    return _run(user_ids, item_ids, user_table.T, item_table.T)


# speedup vs baseline: 1.4534x; 1.4534x over previous
import functools

import jax
import jax.numpy as jnp
from jax import lax
from jax.experimental import pallas as pl
from jax.experimental.pallas import tpu as pltpu
from jax.experimental.pallas import tpu_sc as plsc

L = 16
D = 32
B = 16384
NC = 2
NS = 16
NW = NC * NS
BPW = B // NW
WAVE = 2
WPT = BPW // WAVE
DEPTH = 7
WPG = L // WAVE


def _sc_body(uid_hbm, iid_hbm, ut_hbm, it_hbm, out_hbm,
             uidx_v, iidx_v, ubuf_v, ibuf_v, out_v, usem, isem):
    wid = lax.axis_index("s") * NC + lax.axis_index("c")
    base = wid * BPW

    pltpu.sync_copy(uid_hbm.at[pl.ds(base, BPW)], uidx_v)
    pltpu.sync_copy(iid_hbm.at[pl.ds(base, BPW)], iidx_v)

    iota = lax.iota(jnp.int32, L)

    def vecs(w):
        g = (w // WPG) * L
        return uidx_v[pl.ds(g, L)], iidx_v[pl.ds(g, L)]

    def extract(vec, l):
        return lax.reduce_max(jnp.where(iota == l, vec, 0), (0,))

    def fire(w):
        uvec, ivec = vecs(w)
        p = w % DEPTH
        for s in range(WAVE):
            l = (w % WPG) * WAVE + s
            for vec, tab, buf, sem in ((uvec, ut_hbm, ubuf_v, usem),
                                       (ivec, it_hbm, ibuf_v, isem)):
                sid = extract(vec, l)
                jb = pl.multiple_of((sid >> 7) << 7, 128)
                pltpu.async_copy(tab.at[:, pl.ds(jb, 128)], buf.at[p, s], sem)

    def wait_wave():
        for s in range(WAVE):
            pltpu.make_async_copy(ut_hbm.at[:, pl.ds(0, 128)],
                                  ubuf_v.at[0, s], usem).wait()
            pltpu.make_async_copy(it_hbm.at[:, pl.ds(0, 128)],
                                  ibuf_v.at[0, s], isem).wait()

    d_lo = iota
    d_hi = iota + L

    def compute(w):
        uvec, ivec = vecs(w)
        p = w % DEPTH
        pb = jnp.full((L,), 0, jnp.int32) + p
        for s in range(WAVE):
            l = (w % WPG) * WAVE + s
            sb = jnp.full((L,), s, jnp.int32)
            usid = extract(uvec, l)
            isid = extract(ivec, l)
            ulane = jnp.full((L,), 0, jnp.int32) + (usid & 127)
            ilane = jnp.full((L,), 0, jnp.int32) + (isid & 127)
            u_lo = plsc.load_gather(ubuf_v, [pb, sb, d_lo, ulane])
            u_hi = plsc.load_gather(ubuf_v, [pb, sb, d_hi, ulane])
            i_lo = plsc.load_gather(ibuf_v, [pb, sb, d_lo, ilane])
            i_hi = plsc.load_gather(ibuf_v, [pb, sb, d_hi, ilane])
            prod = u_lo * i_lo + u_hi * i_hi
            score = lax.reduce_sum(prod, (0,))
            k = jnp.full((L,), 0, jnp.int32) + (w * WAVE + s)
            plsc.store_scatter(out_v, [k],
                               jnp.full((L,), 0.0, jnp.float32) + score,
                               mask=iota == 0)

    for w0 in range(DEPTH - 1):
        fire(w0)

    def body(w, carry):
        @pl.when(w + DEPTH - 1 < WPT)
        def _():
            fire(w + DEPTH - 1)
        wait_wave()
        compute(w)
        return carry

    lax.fori_loop(0, WPT, body, 0)

    pltpu.sync_copy(out_v, out_hbm.at[pl.ds(base, BPW)])


@jax.jit
def _run(user_ids, item_ids, user_table_t, item_table_t):
    k = pl.kernel(
        _sc_body,
        out_type=jax.ShapeDtypeStruct((B,), jnp.float32),
        mesh=plsc.VectorSubcoreMesh(core_axis_name="c", subcore_axis_name="s"),
        compiler_params=pltpu.CompilerParams(needs_layout_passes=False),
        scratch_types=[
            pltpu.VMEM((BPW,), jnp.int32),
            pltpu.VMEM((BPW,), jnp.int32),
            pltpu.VMEM((DEPTH, WAVE, D, 128), jnp.float32),
            pltpu.VMEM((DEPTH, WAVE, D, 128), jnp.float32),
            pltpu.VMEM((BPW,), jnp.float32),
            pltpu.SemaphoreType.DMA,
            pltpu.SemaphoreType.DMA,
        ],
    )
    return k(user_ids, item_ids, user_table_t, item_table_t)


def kernel(user_ids, item_ids, user_table, item_table):
    return _run(user_ids, item_ids, user_table.T, item_table.T)
